# Initial kernel scaffold; baseline (speedup 1.0000x reference)
import jax
import jax.numpy as jnp
from jax.experimental import pallas as pl


def kernel(inp, gate_w, gate_b, w1, b1, w2, b2, ln_g, ln_b):
    raise NotImplementedError("write your pallas kernel here")


# speedup vs baseline: 1.3099x; 1.3099x over previous
import functools

import jax
import jax.numpy as jnp
from jax import lax
from jax.experimental import pallas as pl
from jax.experimental.pallas import tpu as pltpu

T = 2048
D = 1024
F = 2048
E = 16
NEG = -1e30


def _gating_body(x_ref, gw_ref, gb_ref, idx_ref, comb_ref):
    x = x_ref[...]
    logits = jnp.dot(x, gw_ref[...], preferred_element_type=jnp.float32)
    logits = logits + gb_ref[...]
    iota = lax.broadcasted_iota(jnp.int32, (T, E), 1)
    m1 = jnp.max(logits, axis=1, keepdims=True)
    am1 = jnp.min(jnp.where(logits == m1, iota, E), axis=1, keepdims=True)
    oh1 = (iota == am1)
    l2 = jnp.where(oh1, NEG, logits)
    m2 = jnp.max(l2, axis=1, keepdims=True)
    am2 = jnp.min(jnp.where(l2 == m2, iota, E), axis=1, keepdims=True)
    oh2 = (iota == am2)
    t = jnp.exp(m2 - m1)
    denom = 1.0 + t
    s1 = 1.0 / denom
    s2 = t / denom
    idx_ref[...] = jnp.concatenate([am1, am2], axis=1)
    comb_ref[...] = jnp.where(oh1, s1, 0.0) + jnp.where(oh2, s2, 0.0)


def _gating(x, gate_w, gate_b):
    return pl.pallas_call(
        _gating_body,
        out_shape=(
            jax.ShapeDtypeStruct((T, 2), jnp.int32),
            jax.ShapeDtypeStruct((T, E), jnp.float32),
        ),
    )(x, gate_w, gate_b.reshape(1, E))


def _ffn_body(x_ref, w1_ref, b1_ref, w2_ref, b2_ref, cT_ref, out_ref, acc_ref):
    e = pl.program_id(0)

    @pl.when(e == 0)
    def _():
        acc_ref[...] = jnp.zeros_like(acc_ref)

    def blk(i, _):
        xb = x_ref[pl.ds(i * 256, 256), :]
        h = jnp.dot(xb, w1_ref[0], preferred_element_type=jnp.float32)
        h = jnp.maximum(h + b1_ref[0], 0.0)
        y = jnp.dot(h, w2_ref[0], preferred_element_type=jnp.float32)
        y = y + b2_ref[0]
        c = cT_ref[0, 0, pl.ds(i * 256, 256)]
        acc_ref[pl.ds(i * 256, 256), :] += c[:, None] * y
        return 0

    lax.fori_loop(0, T // 256, blk, 0)

    @pl.when(e == E - 1)
    def _():
        out_ref[...] = acc_ref[...]


def _ffn(x, w1, b1, w2, b2, combT):
    return pl.pallas_call(
        _ffn_body,
        grid=(E,),
        in_specs=[
            pl.BlockSpec((T, D), lambda e: (0, 0)),
            pl.BlockSpec((1, D, F), lambda e: (e, 0, 0)),
            pl.BlockSpec((1, 1, F), lambda e: (e, 0, 0)),
            pl.BlockSpec((1, F, D), lambda e: (e, 0, 0)),
            pl.BlockSpec((1, 1, D), lambda e: (e, 0, 0)),
            pl.BlockSpec((1, 1, T), lambda e: (e, 0, 0)),
        ],
        out_specs=pl.BlockSpec((T, D), lambda e: (0, 0)),
        out_shape=jax.ShapeDtypeStruct((T, D), jnp.float32),
        scratch_shapes=[pltpu.VMEM((T, D), jnp.float32)],
    )(x, w1, b1.reshape(E, 1, F), w2, b2.reshape(E, 1, D), combT.reshape(E, 1, T))


def _ln_body(x_ref, co_ref, g_ref, b_ref, out_ref):
    v = x_ref[...] + co_ref[...]
    mu = jnp.mean(v, axis=1, keepdims=True)
    d = v - mu
    var = jnp.mean(d * d, axis=1, keepdims=True)
    out_ref[...] = d * lax.rsqrt(var + 1e-5) * g_ref[...] + b_ref[...]


def _ln(x, core_out, ln_g, ln_b):
    return pl.pallas_call(
        _ln_body,
        grid=(8,),
        in_specs=[
            pl.BlockSpec((T // 8, D), lambda i: (i, 0)),
            pl.BlockSpec((T // 8, D), lambda i: (i, 0)),
            pl.BlockSpec((1, D), lambda i: (0, 0)),
            pl.BlockSpec((1, D), lambda i: (0, 0)),
        ],
        out_specs=pl.BlockSpec((T // 8, D), lambda i: (i, 0)),
        out_shape=jax.ShapeDtypeStruct((T, D), jnp.float32),
    )(x, core_out, ln_g.reshape(1, D), ln_b.reshape(1, D))


def kernel(inp, gate_w, gate_b, w1, b1, w2, b2, ln_g, ln_b):
    topk_idx, combine = _gating(inp, gate_w, gate_b)
    core_out = _ffn(inp, w1, b1, w2, b2, combine.T)
    output = _ln(inp, core_out, ln_g, ln_b)
    return output, topk_idx
